# Initial kernel scaffold; baseline (speedup 1.0000x reference)
import jax
import jax.numpy as jnp
from jax.experimental import pallas as pl


def kernel(q_emb, keys, k):
    raise NotImplementedError("write your pallas kernel here")


# speedup vs baseline: 5.6470x; 5.6470x over previous
import functools

import jax
import jax.numpy as jnp
from jax import lax
from jax.experimental import pallas as pl
from jax.experimental.pallas import tpu as pltpu
from jax.experimental.pallas import tpu_sc as plsc

Q = 1024
DIM = 128
N = 100000
TILE = 2048
KT = 49
NPAD = KT * TILE
CHUNK = 128
NCH = NPAD // CHUNK
NCHP = 896
QB = 512
NQB = Q // QB
K = 16
NROWS = Q * K
NW = 32
RPW = NROWS // NW
NG = RPW // 128

_NEG_INF = float("-inf")
_NEG_BIG = -1e30


def _score_kernel(q_ref, keys_ref, scores_ref, cm_ref, cm_acc, qn_scr):
    kt = pl.program_id(1)

    @pl.when(kt == 0)
    def _():
        cm_acc[...] = jnp.full((QB, NCHP), _NEG_BIG, jnp.float32)
        z = q_ref[...]
        zz = z * z
        c = zz[:, 0:8]
        for j in range(1, 16):
            c = c + zz[:, 8 * j:8 * j + 8]
        t1 = c[:, 0:4] + c[:, 4:8]
        t2 = t1[:, 0:2] + t1[:, 2:4]
        ss = t2[:, 0:1] + t2[:, 1:2]
        nrm = lax.rsqrt(ss) * ss
        nrm = jnp.where(ss == 0.0, 0.0, nrm)
        nrm = jnp.maximum(nrm, 1e-12)
        qn_scr[...] = z / nrm

    s = lax.dot_general(qn_scr[...], keys_ref[...], (((1,), (1,)), ((), ())),
                        preferred_element_type=jnp.float32)

    def finish(s):
        scores_ref[...] = s
        cm = jnp.max(s.reshape(QB, K, CHUNK), axis=2)
        cm_pad = jnp.concatenate(
            [cm, jnp.full((QB, NCHP - K), _NEG_BIG, jnp.float32)], axis=1)
        rolled = pltpu.roll(cm_pad, kt * K, axis=1)
        lane = lax.broadcasted_iota(jnp.int32, (QB, NCHP), 1)
        inr = (lane >= kt * K) & (lane < kt * K + K)
        cm_acc[...] = jnp.where(inr, rolled, cm_acc[...])

    @pl.when(kt < KT - 1)
    def _():
        finish(s)

    @pl.when(kt == KT - 1)
    def _():
        col = lax.broadcasted_iota(jnp.int32, (QB, TILE), 1)
        finish(jnp.where(kt * TILE + col < N, s, _NEG_BIG))
        cm_ref[...] = cm_acc[...]


def _select_kernel(cm_ref, sel_ref, rows_ref):
    qb = pl.program_id(0)
    col = lax.broadcasted_iota(jnp.int32, (QB, NCHP), 1)
    v = jnp.where(col < NCH, cm_ref[...], _NEG_INF)
    sels = []
    for _ in range(K):
        m = jnp.max(v, axis=1, keepdims=True)
        cj = jnp.min(jnp.where(v == m, col, jnp.int32(NCHP)),
                     axis=1, keepdims=True)
        sels.append(cj)
        v = jnp.where(col == cj, _NEG_INF, v)
    sel = jnp.concatenate(sels, axis=1)
    sel_ref[...] = sel
    qi = lax.broadcasted_iota(jnp.int32, (QB, K), 0) + qb * QB
    rows_ref[...] = qi * NCH + sel


def _final_kernel(cand_ref, sel_ref, i_ref, p_ref):
    v = cand_ref[...]
    sel = sel_ref[...]
    base = sel * CHUNK
    w = lax.broadcasted_iota(jnp.int32, (QB, CHUNK), 1)
    g = jnp.concatenate(
        [jnp.broadcast_to(base[:, j:j + 1], (QB, CHUNK)) + w
         for j in range(K)], axis=1)
    big = jnp.int32(1 << 30)
    ds, inds = [], []
    for _ in range(K):
        m = jnp.max(v, axis=1, keepdims=True)
        gi = jnp.min(jnp.where(v == m, g, big), axis=1, keepdims=True)
        ds.append(m)
        inds.append(gi)
        v = jnp.where(g == gi, _NEG_INF, v)
    d = jnp.concatenate(ds, axis=1)
    ind = jnp.concatenate(inds, axis=1)
    nrm = jnp.sqrt(jnp.sum(d * d, axis=1, keepdims=True))
    sims = d / jnp.maximum(nrm, 1e-12)
    mx = jnp.max(sims, axis=1, keepdims=True)
    e = jnp.exp(sims - mx)
    i_ref[...] = ind
    p_ref[...] = e / jnp.sum(e, axis=1, keepdims=True)


@functools.cache
def _make_sc_gather():
    @functools.partial(
        pl.kernel,
        out_type=jax.ShapeDtypeStruct((NROWS, CHUNK), jnp.float32),
        mesh=plsc.VectorSubcoreMesh(core_axis_name="c", subcore_axis_name="s"),
        scratch_types=[
            pltpu.VMEM((NG, 128), jnp.int32),
            pltpu.VMEM((RPW, CHUNK), jnp.float32),
            pltpu.SemaphoreType.DMA,
        ],
    )
    def _sc_gather_kernel(scores_hbm, rowidx_hbm, out_hbm, idx_v, rows_v, sem):
        wid = lax.axis_index("s") * 2 + lax.axis_index("c")
        pltpu.sync_copy(rowidx_hbm.at[wid], idx_v)
        copies = []
        for gidx in range(NG):
            copies.append(pltpu.async_copy(
                scores_hbm.at[idx_v.at[gidx]],
                rows_v.at[pl.ds(gidx * 128, 128)], sem))
        for cp in copies:
            cp.wait()
        pltpu.sync_copy(rows_v, out_hbm.at[pl.ds(wid * RPW, RPW)])

    return _sc_gather_kernel


def _sc_gather(scores2d, rows3):
    return _make_sc_gather()(scores2d, rows3)


def kernel(q_emb, keys, k):
    keys_p = jnp.pad(keys, ((0, NPAD - N), (0, 0)))

    scores, cm = pl.pallas_call(
        _score_kernel,
        grid=(NQB, KT),
        in_specs=[
            pl.BlockSpec((QB, DIM), lambda qb, kt: (qb, 0)),
            pl.BlockSpec((TILE, DIM), lambda qb, kt: (kt, 0)),
        ],
        out_specs=[
            pl.BlockSpec((QB, TILE), lambda qb, kt: (qb, kt)),
            pl.BlockSpec((QB, NCHP), lambda qb, kt: (qb, 0)),
        ],
        out_shape=[
            jax.ShapeDtypeStruct((Q, NPAD), jnp.float32),
            jax.ShapeDtypeStruct((Q, NCHP), jnp.float32),
        ],
        scratch_shapes=[pltpu.VMEM((QB, NCHP), jnp.float32),
                        pltpu.VMEM((QB, DIM), jnp.float32)],
    )(q_emb, keys_p)

    sel, rows = pl.pallas_call(
        _select_kernel,
        grid=(NQB,),
        in_specs=[pl.BlockSpec((QB, NCHP), lambda qb: (qb, 0))],
        out_specs=[
            pl.BlockSpec((QB, K), lambda qb: (qb, 0)),
            pl.BlockSpec((QB, K), lambda qb: (qb, 0)),
        ],
        out_shape=[
            jax.ShapeDtypeStruct((Q, K), jnp.int32),
            jax.ShapeDtypeStruct((Q, K), jnp.int32),
        ],
    )(cm)

    cand = _sc_gather(scores.reshape(Q * NCH, CHUNK),
                      rows.reshape(NW, NG, 128))

    ind, probs = pl.pallas_call(
        _final_kernel,
        grid=(NQB,),
        in_specs=[
            pl.BlockSpec((QB, K * CHUNK), lambda qb: (qb, 0)),
            pl.BlockSpec((QB, K), lambda qb: (qb, 0)),
        ],
        out_specs=[
            pl.BlockSpec((QB, K), lambda qb: (qb, 0)),
            pl.BlockSpec((QB, K), lambda qb: (qb, 0)),
        ],
        out_shape=[
            jax.ShapeDtypeStruct((Q, K), jnp.int32),
            jax.ShapeDtypeStruct((Q, K), jnp.float32),
        ],
    )(cand.reshape(Q, K * CHUNK), sel)

    ind = ind + jnp.asarray(k - K, dtype=ind.dtype)
    return (ind, probs)


# speedup vs baseline: 5.6946x; 1.0084x over previous
import functools

import jax
import jax.numpy as jnp
from jax import lax
from jax.experimental import pallas as pl
from jax.experimental.pallas import tpu as pltpu
from jax.experimental.pallas import tpu_sc as plsc

Q = 1024
DIM = 128
N = 100000
TILE = 2048
KT = 49
NPAD = KT * TILE
CHUNK = 128
NCH = NPAD // CHUNK
NCHP = 896
QB = 512
NQB = Q // QB
K = 16
NROWS = Q * K
NW = 32
RPW = NROWS // NW
NG = RPW // 128

_NEG_INF = float("-inf")
_NEG_BIG = -1e30


def _score_kernel(q_ref, keys_ref, scores_ref, sel_ref, rows_ref, cm_acc,
                  qn_scr):
    qb = pl.program_id(0)
    kt = pl.program_id(1)

    @pl.when(kt == 0)
    def _():
        cm_acc[...] = jnp.full((QB, NCHP), _NEG_BIG, jnp.float32)
        z = q_ref[...]
        zz = z * z
        c = zz[:, 0:8]
        for j in range(1, 16):
            c = c + zz[:, 8 * j:8 * j + 8]
        t1 = c[:, 0:4] + c[:, 4:8]
        t2 = t1[:, 0:2] + t1[:, 2:4]
        ss = t2[:, 0:1] + t2[:, 1:2]
        nrm = lax.rsqrt(ss) * ss
        nrm = jnp.where(ss == 0.0, 0.0, nrm)
        nrm = jnp.maximum(nrm, 1e-12)
        qn_scr[...] = z / nrm

    s = lax.dot_general(qn_scr[...], keys_ref[...], (((1,), (1,)), ((), ())),
                        preferred_element_type=jnp.float32)

    def finish(s):
        scores_ref[...] = s
        cm = jnp.max(s.reshape(QB, K, CHUNK), axis=2)
        cm_pad = jnp.concatenate(
            [cm, jnp.full((QB, NCHP - K), _NEG_BIG, jnp.float32)], axis=1)
        rolled = pltpu.roll(cm_pad, kt * K, axis=1)
        lane = lax.broadcasted_iota(jnp.int32, (QB, NCHP), 1)
        inr = (lane >= kt * K) & (lane < kt * K + K)
        cm_acc[...] = jnp.where(inr, rolled, cm_acc[...])

    @pl.when(kt < KT - 1)
    def _():
        finish(s)

    @pl.when(kt == KT - 1)
    def _():
        col = lax.broadcasted_iota(jnp.int32, (QB, TILE), 1)
        finish(jnp.where(kt * TILE + col < N, s, _NEG_BIG))
        cw = lax.broadcasted_iota(jnp.int32, (QB, NCHP), 1)
        v = jnp.where(cw < NCH, cm_acc[...], _NEG_INF)
        sels = []
        for _ in range(K):
            m = jnp.max(v, axis=1, keepdims=True)
            cj = jnp.min(jnp.where(v == m, cw, jnp.int32(NCHP)),
                         axis=1, keepdims=True)
            sels.append(cj)
            v = jnp.where(cw == cj, _NEG_INF, v)
        sel = jnp.concatenate(sels, axis=1)
        sel_ref[...] = sel
        qi = lax.broadcasted_iota(jnp.int32, (QB, K), 0) + qb * QB
        rows_ref[...] = qi * NCH + sel


def _final_kernel(cand_ref, sel_ref, i_ref, p_ref):
    v = cand_ref[...]
    sel = sel_ref[...]
    base = sel * CHUNK
    w = lax.broadcasted_iota(jnp.int32, (QB, CHUNK), 1)
    g = jnp.concatenate(
        [jnp.broadcast_to(base[:, j:j + 1], (QB, CHUNK)) + w
         for j in range(K)], axis=1)
    big = jnp.int32(1 << 30)
    ds, inds = [], []
    for _ in range(K):
        m = jnp.max(v, axis=1, keepdims=True)
        gi = jnp.min(jnp.where(v == m, g, big), axis=1, keepdims=True)
        ds.append(m)
        inds.append(gi)
        v = jnp.where(g == gi, _NEG_INF, v)
    d = jnp.concatenate(ds, axis=1)
    ind = jnp.concatenate(inds, axis=1)
    nrm = jnp.sqrt(jnp.sum(d * d, axis=1, keepdims=True))
    sims = d / jnp.maximum(nrm, 1e-12)
    mx = jnp.max(sims, axis=1, keepdims=True)
    e = jnp.exp(sims - mx)
    i_ref[...] = ind
    p_ref[...] = e / jnp.sum(e, axis=1, keepdims=True)


@functools.cache
def _make_sc_gather():
    @functools.partial(
        pl.kernel,
        out_type=jax.ShapeDtypeStruct((NROWS, CHUNK), jnp.float32),
        mesh=plsc.VectorSubcoreMesh(core_axis_name="c", subcore_axis_name="s"),
        scratch_types=[
            pltpu.VMEM((NG, 128), jnp.int32),
            pltpu.VMEM((RPW, CHUNK), jnp.float32),
            pltpu.SemaphoreType.DMA,
        ],
    )
    def _sc_gather_kernel(scores_hbm, rowidx_hbm, out_hbm, idx_v, rows_v, sem):
        wid = lax.axis_index("s") * 2 + lax.axis_index("c")
        pltpu.sync_copy(rowidx_hbm.at[wid], idx_v)
        copies = []
        for gidx in range(NG):
            copies.append(pltpu.async_copy(
                scores_hbm.at[idx_v.at[gidx]],
                rows_v.at[pl.ds(gidx * 128, 128)], sem))
        for cp in copies:
            cp.wait()
        pltpu.sync_copy(rows_v, out_hbm.at[pl.ds(wid * RPW, RPW)])

    return _sc_gather_kernel


def _sc_gather(scores2d, rows3):
    return _make_sc_gather()(scores2d, rows3)


def kernel(q_emb, keys, k):
    keys_p = jnp.pad(keys, ((0, NPAD - N), (0, 0)))

    scores, sel, rows = pl.pallas_call(
        _score_kernel,
        grid=(NQB, KT),
        in_specs=[
            pl.BlockSpec((QB, DIM), lambda qb, kt: (qb, 0)),
            pl.BlockSpec((TILE, DIM), lambda qb, kt: (kt, 0)),
        ],
        out_specs=[
            pl.BlockSpec((QB, TILE), lambda qb, kt: (qb, kt)),
            pl.BlockSpec((QB, K), lambda qb, kt: (qb, 0)),
            pl.BlockSpec((QB, K), lambda qb, kt: (qb, 0)),
        ],
        out_shape=[
            jax.ShapeDtypeStruct((Q, NPAD), jnp.float32),
            jax.ShapeDtypeStruct((Q, K), jnp.int32),
            jax.ShapeDtypeStruct((Q, K), jnp.int32),
        ],
        scratch_shapes=[pltpu.VMEM((QB, NCHP), jnp.float32),
                        pltpu.VMEM((QB, DIM), jnp.float32)],
    )(q_emb, keys_p)

    cand = _sc_gather(scores.reshape(Q * NCH, CHUNK),
                      rows.reshape(NW, NG, 128))

    ind, probs = pl.pallas_call(
        _final_kernel,
        grid=(NQB,),
        in_specs=[
            pl.BlockSpec((QB, K * CHUNK), lambda qb: (qb, 0)),
            pl.BlockSpec((QB, K), lambda qb: (qb, 0)),
        ],
        out_specs=[
            pl.BlockSpec((QB, K), lambda qb: (qb, 0)),
            pl.BlockSpec((QB, K), lambda qb: (qb, 0)),
        ],
        out_shape=[
            jax.ShapeDtypeStruct((Q, K), jnp.int32),
            jax.ShapeDtypeStruct((Q, K), jnp.float32),
        ],
    )(cand.reshape(Q, K * CHUNK), sel)

    ind = ind + jnp.asarray(k - K, dtype=ind.dtype)
    return (ind, probs)


# speedup vs baseline: 5.7519x; 1.0100x over previous
import functools

import jax
import jax.numpy as jnp
from jax import lax
from jax.experimental import pallas as pl
from jax.experimental.pallas import tpu as pltpu
from jax.experimental.pallas import tpu_sc as plsc

Q = 1024
DIM = 128
N = 100000
TILE = 3584
KT = 28
NPAD = KT * TILE
CHUNK = 128
CPT = TILE // CHUNK
NCH = NPAD // CHUNK
NCHP = 896
QB = 512
NQB = Q // QB
K = 16
NROWS = Q * K
NW = 32
RPW = NROWS // NW
NG = RPW // 128

_NEG_INF = float("-inf")
_NEG_BIG = -1e30


def _score_kernel(q_ref, keys_ref, scores_ref, sel_ref, rows_ref, cm_acc,
                  qn_scr):
    qb = pl.program_id(0)
    kt = pl.program_id(1)

    @pl.when(kt == 0)
    def _():
        cm_acc[...] = jnp.full((QB, NCHP), _NEG_BIG, jnp.float32)
        z = q_ref[...]
        zz = z * z
        c = zz[:, 0:8]
        for j in range(1, 16):
            c = c + zz[:, 8 * j:8 * j + 8]
        t1 = c[:, 0:4] + c[:, 4:8]
        t2 = t1[:, 0:2] + t1[:, 2:4]
        ss = t2[:, 0:1] + t2[:, 1:2]
        nrm = lax.rsqrt(ss) * ss
        nrm = jnp.where(ss == 0.0, 0.0, nrm)
        nrm = jnp.maximum(nrm, 1e-12)
        qn_scr[...] = z / nrm

    s = lax.dot_general(qn_scr[...], keys_ref[...], (((1,), (1,)), ((), ())),
                        preferred_element_type=jnp.float32)

    def finish(s):
        scores_ref[...] = s
        cm = jnp.max(s.reshape(QB, CPT, CHUNK), axis=2)
        cm_pad = jnp.concatenate(
            [cm, jnp.full((QB, NCHP - CPT), _NEG_BIG, jnp.float32)], axis=1)
        rolled = pltpu.roll(cm_pad, kt * CPT, axis=1)
        lane = lax.broadcasted_iota(jnp.int32, (QB, NCHP), 1)
        inr = (lane >= kt * CPT) & (lane < kt * CPT + CPT)
        cm_acc[...] = jnp.where(inr, rolled, cm_acc[...])

    @pl.when(kt < KT - 1)
    def _():
        finish(s)

    @pl.when(kt == KT - 1)
    def _():
        col = lax.broadcasted_iota(jnp.int32, (QB, TILE), 1)
        finish(jnp.where(kt * TILE + col < N, s, _NEG_BIG))
        cw = lax.broadcasted_iota(jnp.int32, (QB, NCHP), 1)
        v = jnp.where(cw < NCH, cm_acc[...], _NEG_INF)
        sels = []
        for _ in range(K):
            m = jnp.max(v, axis=1, keepdims=True)
            cj = jnp.min(jnp.where(v == m, cw, jnp.int32(NCHP)),
                         axis=1, keepdims=True)
            sels.append(cj)
            v = jnp.where(cw == cj, _NEG_INF, v)
        sel = jnp.concatenate(sels, axis=1)
        sel_ref[...] = sel
        qi = lax.broadcasted_iota(jnp.int32, (QB, K), 0) + qb * QB
        rows_ref[...] = qi * NCH + sel


def _final_kernel(cand_ref, sel_ref, i_ref, p_ref):
    v = cand_ref[...]
    sel = sel_ref[...]
    base = sel * CHUNK
    w = lax.broadcasted_iota(jnp.int32, (QB, CHUNK), 1)
    g = jnp.concatenate(
        [jnp.broadcast_to(base[:, j:j + 1], (QB, CHUNK)) + w
         for j in range(K)], axis=1)
    big = jnp.int32(1 << 30)
    ds, inds = [], []
    for _ in range(K):
        m = jnp.max(v, axis=1, keepdims=True)
        gi = jnp.min(jnp.where(v == m, g, big), axis=1, keepdims=True)
        ds.append(m)
        inds.append(gi)
        v = jnp.where(g == gi, _NEG_INF, v)
    d = jnp.concatenate(ds, axis=1)
    ind = jnp.concatenate(inds, axis=1)
    nrm = jnp.sqrt(jnp.sum(d * d, axis=1, keepdims=True))
    sims = d / jnp.maximum(nrm, 1e-12)
    mx = jnp.max(sims, axis=1, keepdims=True)
    e = jnp.exp(sims - mx)
    i_ref[...] = ind
    p_ref[...] = e / jnp.sum(e, axis=1, keepdims=True)


@functools.cache
def _make_sc_gather():
    @functools.partial(
        pl.kernel,
        out_type=jax.ShapeDtypeStruct((NROWS, CHUNK), jnp.float32),
        mesh=plsc.VectorSubcoreMesh(core_axis_name="c", subcore_axis_name="s"),
        scratch_types=[
            pltpu.VMEM((NG, 128), jnp.int32),
            pltpu.VMEM((RPW, CHUNK), jnp.float32),
            pltpu.SemaphoreType.DMA,
        ],
    )
    def _sc_gather_kernel(scores_hbm, rowidx_hbm, out_hbm, idx_v, rows_v, sem):
        wid = lax.axis_index("s") * 2 + lax.axis_index("c")
        pltpu.sync_copy(rowidx_hbm.at[wid], idx_v)
        copies = []
        for gidx in range(NG):
            copies.append(pltpu.async_copy(
                scores_hbm.at[idx_v.at[gidx]],
                rows_v.at[pl.ds(gidx * 128, 128)], sem))
        for cp in copies:
            cp.wait()
        pltpu.sync_copy(rows_v, out_hbm.at[pl.ds(wid * RPW, RPW)])

    return _sc_gather_kernel


def _sc_gather(scores2d, rows3):
    return _make_sc_gather()(scores2d, rows3)


def kernel(q_emb, keys, k):
    keys_p = jnp.pad(keys, ((0, NPAD - N), (0, 0)))

    scores, sel, rows = pl.pallas_call(
        _score_kernel,
        grid=(NQB, KT),
        in_specs=[
            pl.BlockSpec((QB, DIM), lambda qb, kt: (qb, 0)),
            pl.BlockSpec((TILE, DIM), lambda qb, kt: (kt, 0)),
        ],
        out_specs=[
            pl.BlockSpec((QB, TILE), lambda qb, kt: (qb, kt)),
            pl.BlockSpec((QB, K), lambda qb, kt: (qb, 0)),
            pl.BlockSpec((QB, K), lambda qb, kt: (qb, 0)),
        ],
        out_shape=[
            jax.ShapeDtypeStruct((Q, NPAD), jnp.float32),
            jax.ShapeDtypeStruct((Q, K), jnp.int32),
            jax.ShapeDtypeStruct((Q, K), jnp.int32),
        ],
        scratch_shapes=[pltpu.VMEM((QB, NCHP), jnp.float32),
                        pltpu.VMEM((QB, DIM), jnp.float32)],
    )(q_emb, keys_p)

    cand = _sc_gather(scores.reshape(Q * NCH, CHUNK),
                      rows.reshape(NW, NG, 128))

    ind, probs = pl.pallas_call(
        _final_kernel,
        grid=(NQB,),
        in_specs=[
            pl.BlockSpec((QB, K * CHUNK), lambda qb: (qb, 0)),
            pl.BlockSpec((QB, K), lambda qb: (qb, 0)),
        ],
        out_specs=[
            pl.BlockSpec((QB, K), lambda qb: (qb, 0)),
            pl.BlockSpec((QB, K), lambda qb: (qb, 0)),
        ],
        out_shape=[
            jax.ShapeDtypeStruct((Q, K), jnp.int32),
            jax.ShapeDtypeStruct((Q, K), jnp.float32),
        ],
    )(cand.reshape(Q, K * CHUNK), sel)

    ind = ind + jnp.asarray(k - K, dtype=ind.dtype)
    return (ind, probs)
